# baseline (speedup 1.0000x reference)
import functools

import jax
import jax.numpy as jnp
from jax import lax
from jax.experimental import pallas as pl
from jax.experimental.pallas import tpu as pltpu
from jax.experimental.pallas import tpu_sc as plsc

D_MODEL = 1024
D_FFN = 2048
FT = 1024
N_EXPERTS = 8
BLOCK = 2048
BATCH = 2
K = 512
NP = BATCH * N_EXPERTS
ROWS = NP * K
RPW = ROWS // 32
CH = 64


def _router_body(choice_ref, xt_ref, slots_ref, g_ref):
    b = pl.program_id(0)
    ch = choice_ref[...]
    xt = xt_ref[0]
    lt = jnp.dot(ch, xt, preferred_element_type=jnp.float32)
    m = jnp.max(lt, axis=0, keepdims=True)
    ex = jnp.exp(lt - m)
    p = ex / jnp.sum(ex, axis=0, keepdims=True)
    bits = lax.bitcast_convert_type(p, jnp.int32)

    def vstep(_, lohi):
        lo, hi = lohi
        mid = lo + (hi - lo) // 2
        cnt = jnp.sum((bits >= mid).astype(jnp.int32), axis=1, keepdims=True)
        ge = cnt >= K
        return jnp.where(ge, mid, lo), jnp.where(ge, hi, mid)

    lo0 = jnp.zeros((N_EXPERTS, 1), jnp.int32)
    hi0 = jnp.full((N_EXPERTS, 1), 0x7F800000, jnp.int32)
    v, _ = lax.fori_loop(0, 31, vstep, (lo0, hi0))

    gt = bits > v
    eq = bits == v
    n_gt = jnp.sum(gt.astype(jnp.int32), axis=1, keepdims=True)
    quota = K - n_gt
    idx = lax.broadcasted_iota(jnp.int32, (N_EXPERTS, BLOCK), 1)

    def istep(_, lohi):
        lo, hi = lohi
        mid = lo + (hi - lo) // 2
        c2 = jnp.sum((eq & (idx < mid)).astype(jnp.int32), axis=1, keepdims=True)
        ge = c2 >= quota
        return jnp.where(ge, lo, mid), jnp.where(ge, mid, hi)

    _, m_idx = lax.fori_loop(
        0, 11, istep,
        (jnp.zeros((N_EXPERTS, 1), jnp.int32),
         jnp.full((N_EXPERTS, 1), BLOCK, jnp.int32)))

    keep = gt | (eq & (idx < m_idx))
    gates = jnp.where(keep, p, 0.0)

    i128a = lax.broadcasted_iota(jnp.int32, (128, 128), 0)
    i128b = lax.broadcasted_iota(jnp.int32, (128, 128), 1)
    U128 = (i128a <= i128b).astype(jnp.float32)
    i16a = lax.broadcasted_iota(jnp.int32, (16, 16), 0)
    i16b = lax.broadcasted_iota(jnp.int32, (16, 16), 1)
    SL16 = (i16b < i16a).astype(jnp.float32)
    iota_k = lax.broadcasted_iota(jnp.int32, (K, 1), 0)
    s_row = lax.broadcasted_iota(jnp.int32, (1, BLOCK), 1)
    rhs = jnp.concatenate(
        [(s_row // 256).astype(jnp.float32).reshape(BLOCK, 1),
         (s_row % 256).astype(jnp.float32).reshape(BLOCK, 1)], axis=1)

    slist, glist = [], []
    for e in range(N_EXPERTS):
        keep_row = keep[e:e + 1, :].astype(jnp.float32)
        keep2 = keep_row.reshape(16, 128)
        within = jnp.dot(keep2, U128, preferred_element_type=jnp.float32)
        rowtot = within[:, 127:128]
        rowoff = jnp.dot(SL16, rowtot, preferred_element_type=jnp.float32,
                         precision=jax.lax.Precision.HIGHEST)
        rank0 = (within + rowoff - 1.0).astype(jnp.int32)
        rank_row = rank0.reshape(1, BLOCK)
        Pe = jnp.where((iota_k == rank_row) & (keep_row > 0.0),
                       1.0, 0.0)
        rhs_e = jnp.concatenate([rhs, gates[e].reshape(BLOCK, 1)], axis=1)
        cols = jnp.dot(Pe, rhs_e, preferred_element_type=jnp.float32)
        sl = (cols[:, 0:1] * 256.0 + cols[:, 1:2]).astype(jnp.int32)
        slist.append((sl + b * BLOCK).reshape(1, 1, K))
        glist.append(cols[:, 2:3].reshape(1, K, 1))
    slots_ref[0] = jnp.concatenate(slist, axis=0)
    g_ref[0] = jnp.concatenate(glist, axis=0)


def _router(x, choice):
    xt = jnp.swapaxes(x, 1, 2)
    return pl.pallas_call(
        _router_body,
        grid=(BATCH,),
        in_specs=[
            pl.BlockSpec((N_EXPERTS, D_MODEL), lambda b: (0, 0)),
            pl.BlockSpec((1, D_MODEL, BLOCK), lambda b: (b, 0, 0)),
        ],
        out_specs=[
            pl.BlockSpec((1, N_EXPERTS, 1, K), lambda b: (b, 0, 0, 0)),
            pl.BlockSpec((1, N_EXPERTS, K, 1), lambda b: (b, 0, 0, 0)),
        ],
        out_shape=[
            jax.ShapeDtypeStruct((BATCH, N_EXPERTS, 1, K), jnp.int32),
            jax.ShapeDtypeStruct((BATCH, N_EXPERTS, K, 1), jnp.float32),
        ],
    )(choice, xt)


_MESH = plsc.VectorSubcoreMesh(
    core_axis_name="c", subcore_axis_name="s", num_cores=2, num_subcores=16)


@functools.partial(
    pl.kernel,
    out_type=jax.ShapeDtypeStruct((ROWS, D_MODEL), jnp.float32),
    mesh=_MESH,
    scratch_types=[
        pltpu.VMEM((RPW,), jnp.int32),
        pltpu.VMEM((CH, D_MODEL), jnp.float32),
        pltpu.SemaphoreType.DMA,
    ],
)
def _gather(slots_hbm, x2d_hbm, xin_hbm, myidx, rowbuf, sem):
    c = lax.axis_index("c")
    sid = lax.axis_index("s")
    p2 = c * N_EXPERTS + sid // 2
    half = sid % 2
    pltpu.sync_copy(slots_hbm.at[p2, pl.ds(half * RPW, RPW)], myidx)
    row0 = p2 * K + half * RPW
    for j in range(RPW // CH):
        pltpu.async_copy(x2d_hbm.at[myidx.at[pl.ds(j * CH, CH)]],
                         rowbuf, sem).wait()
        pltpu.sync_copy(rowbuf, xin_hbm.at[pl.ds(row0 + j * CH, CH)])


def _mlp_body(x_ref, g_ref, sl_ref, w1_ref, w2_ref, out_ref):
    b = pl.program_id(0)
    e = pl.program_id(1)

    @pl.when(e == 0)
    def _():
        out_ref[...] = jnp.zeros_like(out_ref)

    x = x_ref[0]
    w1 = w1_ref[0]
    w2 = w2_ref[0]
    g = g_ref[0, 0]
    h = jnp.dot(x.astype(jnp.bfloat16), w1,
                preferred_element_type=jnp.float32)
    h = h * (1.0 / (1.0 + jnp.exp(-h)))
    y = (h * g).astype(jnp.bfloat16)
    ymlp = jnp.dot(y, w2, preferred_element_type=jnp.float32)
    sl = sl_ref[0, 0]
    iota_s = lax.broadcasted_iota(jnp.int32, (BLOCK, 1), 0) + b * BLOCK
    Pt = jnp.where(iota_s == sl, 1.0, 0.0).astype(jnp.bfloat16)
    out_ref[0] += jnp.dot(Pt, ymlp.astype(jnp.bfloat16),
                          preferred_element_type=jnp.float32)


def _mlp_scatter(x_in, gcomp, slots, w1, w2):
    x3 = x_in.reshape(NP, K, D_MODEL)
    w1b = w1.astype(jnp.bfloat16)
    w2b = w2.astype(jnp.bfloat16)
    return pl.pallas_call(
        _mlp_body,
        grid=(BATCH, N_EXPERTS),
        in_specs=[
            pl.BlockSpec((1, K, D_MODEL), lambda b, e: (b * N_EXPERTS + e, 0, 0)),
            pl.BlockSpec((1, 1, K, 1), lambda b, e: (b, e, 0, 0)),
            pl.BlockSpec((1, 1, 1, K), lambda b, e: (b, e, 0, 0)),
            pl.BlockSpec((1, D_MODEL, D_FFN), lambda b, e: (e, 0, 0)),
            pl.BlockSpec((1, D_FFN, D_MODEL), lambda b, e: (e, 0, 0)),
        ],
        out_specs=pl.BlockSpec((1, BLOCK, D_MODEL), lambda b, e: (b, 0, 0)),
        out_shape=jax.ShapeDtypeStruct((BATCH, BLOCK, D_MODEL), jnp.float32),
    )(x3, gcomp, slots, w1b, w2b)


def kernel(x, choice, w1, w2):
    slots, gcomp = _router(x, choice)
    x2d = x.reshape(BATCH * BLOCK, D_MODEL)
    x_in = _gather(slots.reshape(NP, K), x2d)
    return _mlp_scatter(x_in, gcomp, slots, w1, w2)


# speedup vs baseline: 1.1347x; 1.1347x over previous
import functools

import jax
import jax.numpy as jnp
from jax import lax
from jax.experimental import pallas as pl
from jax.experimental.pallas import tpu as pltpu
from jax.experimental.pallas import tpu_sc as plsc

D_MODEL = 1024
D_FFN = 2048
FT = 1024
N_EXPERTS = 8
BLOCK = 2048
BATCH = 2
K = 512
NP = BATCH * N_EXPERTS
ROWS = NP * K
RPW = ROWS // 32
CH = 64


def _router_body(choice_ref, xt_ref, slots_ref, g_ref):
    b = pl.program_id(0)
    ch = choice_ref[...]
    xt = xt_ref[0]
    lt = jnp.dot(ch, xt, preferred_element_type=jnp.float32)
    m = jnp.max(lt, axis=0, keepdims=True)
    ex = jnp.exp(lt - m)
    p = ex / jnp.sum(ex, axis=0, keepdims=True)
    bits = lax.bitcast_convert_type(p, jnp.int32)

    def vstep(_, lohi):
        lo, hi = lohi
        mid = lo + (hi - lo) // 2
        cnt = jnp.sum((bits >= mid).astype(jnp.int32), axis=1, keepdims=True)
        ge = cnt >= K
        return jnp.where(ge, mid, lo), jnp.where(ge, hi, mid)

    lo0 = jnp.zeros((N_EXPERTS, 1), jnp.int32)
    hi0 = jnp.full((N_EXPERTS, 1), 0x7F800000, jnp.int32)
    v, _ = lax.fori_loop(0, 31, vstep, (lo0, hi0))

    gt = bits > v
    eq = bits == v
    n_gt = jnp.sum(gt.astype(jnp.int32), axis=1, keepdims=True)
    quota = K - n_gt
    idx = lax.broadcasted_iota(jnp.int32, (N_EXPERTS, BLOCK), 1)

    def istep(_, lohi):
        lo, hi = lohi
        mid = lo + (hi - lo) // 2
        c2 = jnp.sum((eq & (idx < mid)).astype(jnp.int32), axis=1, keepdims=True)
        ge = c2 >= quota
        return jnp.where(ge, lo, mid), jnp.where(ge, mid, hi)

    _, m_idx = lax.fori_loop(
        0, 11, istep,
        (jnp.zeros((N_EXPERTS, 1), jnp.int32),
         jnp.full((N_EXPERTS, 1), BLOCK, jnp.int32)))

    keep = gt | (eq & (idx < m_idx))
    gates = jnp.where(keep, p, 0.0)

    i128a = lax.broadcasted_iota(jnp.int32, (128, 128), 0)
    i128b = lax.broadcasted_iota(jnp.int32, (128, 128), 1)
    U128 = (i128a <= i128b).astype(jnp.float32)
    i16a = lax.broadcasted_iota(jnp.int32, (16, 16), 0)
    i16b = lax.broadcasted_iota(jnp.int32, (16, 16), 1)
    SL16 = (i16b < i16a).astype(jnp.float32)
    iota_k = lax.broadcasted_iota(jnp.int32, (K, 1), 0)
    s_row = lax.broadcasted_iota(jnp.int32, (1, BLOCK), 1)
    rhs = jnp.concatenate(
        [(s_row // 256).astype(jnp.float32).reshape(BLOCK, 1),
         (s_row % 256).astype(jnp.float32).reshape(BLOCK, 1)], axis=1)

    slist, glist = [], []
    for e in range(N_EXPERTS):
        keep_row = keep[e:e + 1, :].astype(jnp.float32)
        keep2 = keep_row.reshape(16, 128)
        within = jnp.dot(keep2, U128, preferred_element_type=jnp.float32)
        rowtot = within[:, 127:128]
        rowoff = jnp.dot(SL16, rowtot, preferred_element_type=jnp.float32,
                         precision=jax.lax.Precision.HIGHEST)
        rank0 = (within + rowoff - 1.0).astype(jnp.int32)
        rank_row = rank0.reshape(1, BLOCK)
        Pe = jnp.where((iota_k == rank_row) & (keep_row > 0.0),
                       1.0, 0.0)
        rhs_e = jnp.concatenate([rhs, gates[e].reshape(BLOCK, 1)], axis=1)
        cols = jnp.dot(Pe, rhs_e, preferred_element_type=jnp.float32)
        sl = (cols[:, 0:1] * 256.0 + cols[:, 1:2]).astype(jnp.int32)
        slist.append((sl + b * BLOCK).reshape(1, 1, K))
        glist.append(cols[:, 2:3].reshape(1, K, 1))
    slots_ref[0] = jnp.concatenate(slist, axis=0)
    g_ref[0] = jnp.concatenate(glist, axis=0)


def _router(x, choice):
    xt = jnp.swapaxes(x, 1, 2)
    return pl.pallas_call(
        _router_body,
        grid=(BATCH,),
        in_specs=[
            pl.BlockSpec((N_EXPERTS, D_MODEL), lambda b: (0, 0)),
            pl.BlockSpec((1, D_MODEL, BLOCK), lambda b: (b, 0, 0)),
        ],
        out_specs=[
            pl.BlockSpec((1, N_EXPERTS, 1, K), lambda b: (b, 0, 0, 0)),
            pl.BlockSpec((1, N_EXPERTS, K, 1), lambda b: (b, 0, 0, 0)),
        ],
        out_shape=[
            jax.ShapeDtypeStruct((BATCH, N_EXPERTS, 1, K), jnp.int32),
            jax.ShapeDtypeStruct((BATCH, N_EXPERTS, K, 1), jnp.float32),
        ],
    )(choice, xt)


_MESH = plsc.VectorSubcoreMesh(
    core_axis_name="c", subcore_axis_name="s", num_cores=2, num_subcores=16)


@functools.partial(
    pl.kernel,
    out_type=jax.ShapeDtypeStruct((ROWS, D_MODEL), jnp.float32),
    mesh=_MESH,
    scratch_types=[
        pltpu.VMEM((RPW,), jnp.int32),
        pltpu.VMEM((CH, D_MODEL), jnp.float32),
        pltpu.SemaphoreType.DMA,
    ],
)
def _gather(slots_hbm, x2d_hbm, xin_hbm, myidx, rowbuf, sem):
    c = lax.axis_index("c")
    sid = lax.axis_index("s")
    p2 = c * N_EXPERTS + sid // 2
    half = sid % 2
    pltpu.sync_copy(slots_hbm.at[p2, pl.ds(half * RPW, RPW)], myidx)
    row0 = p2 * K + half * RPW
    for j in range(RPW // CH):
        pltpu.async_copy(x2d_hbm.at[myidx.at[pl.ds(j * CH, CH)]],
                         rowbuf, sem).wait()
        pltpu.sync_copy(rowbuf, xin_hbm.at[pl.ds(row0 + j * CH, CH)])


def _mlp_body(x_ref, g_ref, sl_ref, w1_ref, w2_ref, out_ref, yacc):
    b = pl.program_id(0)
    e = pl.program_id(1)
    t = pl.program_id(2)

    @pl.when((e == 0) & (t == 0))
    def _():
        out_ref[...] = jnp.zeros_like(out_ref)

    x = x_ref[0]
    w1 = w1_ref[0]
    w2 = w2_ref[0]
    h = jnp.dot(x.astype(jnp.bfloat16), w1.astype(jnp.bfloat16),
                preferred_element_type=jnp.float32)
    h = h * (1.0 / (1.0 + jnp.exp(-h)))
    contrib = jnp.dot(h.astype(jnp.bfloat16), w2.astype(jnp.bfloat16),
                      preferred_element_type=jnp.float32)

    @pl.when(t == 0)
    def _():
        yacc[...] = contrib

    @pl.when(t > 0)
    def _():
        yacc[...] += contrib

    @pl.when(t == D_FFN // FT - 1)
    def _():
        g = g_ref[0, 0]
        y = (yacc[...] * g).astype(jnp.bfloat16)
        sl = sl_ref[0, 0]
        iota_s = (lax.broadcasted_iota(jnp.int32, (BLOCK, 1), 0)
                  + b * BLOCK)
        Pt = jnp.where(iota_s == sl, 1.0, 0.0).astype(jnp.bfloat16)
        out_ref[0] += jnp.dot(Pt, y, preferred_element_type=jnp.float32)


def _mlp_scatter(x_in, gcomp, slots, w1, w2):
    x3 = x_in.reshape(NP, K, D_MODEL)
    return pl.pallas_call(
        _mlp_body,
        grid=(BATCH, N_EXPERTS, D_FFN // FT),
        in_specs=[
            pl.BlockSpec((1, K, D_MODEL), lambda b, e, t: (b * N_EXPERTS + e, 0, 0)),
            pl.BlockSpec((1, 1, K, 1), lambda b, e, t: (b, e, 0, 0)),
            pl.BlockSpec((1, 1, 1, K), lambda b, e, t: (b, e, 0, 0)),
            pl.BlockSpec((1, D_MODEL, FT), lambda b, e, t: (e, 0, t)),
            pl.BlockSpec((1, FT, D_MODEL), lambda b, e, t: (e, t, 0)),
        ],
        out_specs=pl.BlockSpec((1, BLOCK, D_MODEL), lambda b, e, t: (b, 0, 0)),
        out_shape=jax.ShapeDtypeStruct((BATCH, BLOCK, D_MODEL), jnp.float32),
        scratch_shapes=[pltpu.VMEM((K, D_MODEL), jnp.float32)],
    )(x3, gcomp, slots, w1, w2)


def kernel(x, choice, w1, w2):
    slots, gcomp = _router(x, choice)
    x2d = x.reshape(BATCH * BLOCK, D_MODEL)
    x_in = _gather(slots.reshape(NP, K), x2d)
    return _mlp_scatter(x_in, gcomp, slots, w1, w2)
